# baseline (speedup 1.0000x reference)
import functools

import jax
import jax.numpy as jnp
from jax import lax
from jax.experimental import pallas as pl
from jax.experimental.pallas import tpu as pltpu
from jax.experimental.pallas import tpu_sc as plsc

BATCH = 16384
DF = 64
DW = 128

_NC = 2
_NS = 16
_NW = _NC * _NS
_BPW = BATCH // _NW
_CH = 128
_NCH = _BPW // _CH
_HALF = _BPW // 2


_RBLK = 2000


def _repack_body(in_ref, out_ref):
    x = in_ref[...]
    x3 = x.reshape(_RBLK, 2, DF)
    out_ref[...] = jnp.concatenate([x3[:, 0, :], x3[:, 1, :]], axis=-1)


def _repack(table):
    n2 = table.shape[0] // 2
    return pl.pallas_call(
        _repack_body,
        grid=(n2 // _RBLK,),
        in_specs=[pl.BlockSpec((2 * _RBLK, DF), lambda i: (i, 0))],
        out_specs=pl.BlockSpec((_RBLK, DW), lambda i: (i, 0)),
        out_shape=jax.ShapeDtypeStruct((n2, DW), jnp.float32),
    )(table)


@functools.cache
def _make_sc_gather():
    mesh = plsc.VectorSubcoreMesh(core_axis_name="c", subcore_axis_name="s")

    def _body(idx_hbm, tab_hbm, out_hbm, idx_v, rows_v, gsem, wsem):
        wid = lax.axis_index("s") * _NC + lax.axis_index("c")
        base = wid * _BPW
        pltpu.sync_copy(idx_hbm.at[pl.ds(wid * _NCH, _NCH)], idx_v)
        writes = []
        for p in range(2):
            if p:
                for w in writes:
                    w.wait()
                writes = []
            gathers = []
            for j in range(2):
                c = 2 * p + j
                gathers.append(pltpu.async_copy(
                    tab_hbm.at[idx_v.at[c]],
                    rows_v.at[pl.ds(j * _CH, _CH)], gsem))
            for g in gathers:
                g.wait()
            writes.append(pltpu.async_copy(
                rows_v, out_hbm.at[pl.ds(base + p * _HALF, _HALF)], wsem))
        for w in writes:
            w.wait()

    return pl.kernel(
        _body,
        mesh=mesh,
        out_type=jax.ShapeDtypeStruct((BATCH, DW), jnp.float32),
        scratch_types=[
            pltpu.VMEM((_NCH, _CH), jnp.int32),
            pltpu.VMEM((_HALF, DW), jnp.float32),
            pltpu.SemaphoreType.DMA,
            pltpu.SemaphoreType.DMA,
        ],
    )


_BLK = 1024


def _mlp_body(u_ref, s_ref, up_ref, sp_ref, w0u_ref, w0s_ref, b0_ref,
              w1_ref, b1_ref, w2_ref, b2_ref, w3_ref, b3_ref, out_ref):
    u2 = u_ref[...]
    s2 = s_ref[...]
    u = jnp.where(up_ref[...] != 0, u2[:, DF:], u2[:, :DF])
    s = jnp.where(sp_ref[...] != 0, s2[:, DF:], s2[:, :DF])
    x = (jnp.dot(u, w0u_ref[...], preferred_element_type=jnp.float32)
         + jnp.dot(s, w0s_ref[...], preferred_element_type=jnp.float32)
         + b0_ref[...])
    x = jnp.maximum(x, 0.0)
    x = jnp.dot(x, w1_ref[...], preferred_element_type=jnp.float32) + b1_ref[...]
    x = jnp.maximum(x, 0.0)
    x = jnp.dot(x, w2_ref[...], preferred_element_type=jnp.float32) + b2_ref[...]
    x = jnp.maximum(x, 0.0)
    out_ref[...] = jnp.sum(x * w3_ref[...], axis=1) + b3_ref[0, 0]


def _mlp(u, s, up, sp, w0u, w0s, b0, w1, b1, w2, b2, w3, b3):
    grid = (BATCH // _BLK,)
    full = lambda shape: pl.BlockSpec(shape, lambda i: (0,) * len(shape))
    return pl.pallas_call(
        _mlp_body,
        grid=grid,
        in_specs=[
            pl.BlockSpec((_BLK, DW), lambda i: (i, 0)),
            pl.BlockSpec((_BLK, DW), lambda i: (i, 0)),
            pl.BlockSpec((_BLK, 1), lambda i: (i, 0)),
            pl.BlockSpec((_BLK, 1), lambda i: (i, 0)),
            full((DF, 128)),
            full((DF, 128)),
            full((1, 128)),
            full((128, 64)),
            full((1, 64)),
            full((64, 32)),
            full((1, 32)),
            full((1, 32)),
            full((1, 1)),
        ],
        out_specs=pl.BlockSpec((_BLK,), lambda i: (i,)),
        out_shape=jax.ShapeDtypeStruct((BATCH,), jnp.float32),
    )(u, s, up, sp, w0u, w0s, b0, w1, b1, w2, b2, w3, b3)


def kernel(user_ids, song_ids, user_table, song_table,
           W0, b0, W1, b1, W2, b2, W3, b3):
    uid = user_ids.astype(jnp.int32)
    sid = song_ids.astype(jnp.int32)
    uidx = (uid >> 1).reshape(_NW * _NCH, _CH)
    sidx = (sid >> 1).reshape(_NW * _NCH, _CH)
    upar = (uid & 1).reshape(BATCH, 1)
    spar = (sid & 1).reshape(BATCH, 1)
    st2 = _repack(song_table)
    ut2 = _repack(user_table)
    gather = _make_sc_gather()
    s = gather(sidx, st2)
    u = gather(uidx, ut2)
    w0t = W0.T
    return _mlp(
        u, s, upar, spar,
        w0t[:DF], w0t[DF:], b0.reshape(1, 128),
        W1.T, b1.reshape(1, 64),
        W2.T, b2.reshape(1, 32),
        W3, b3.reshape(1, 1),
    )


# speedup vs baseline: 1.0037x; 1.0037x over previous
import functools

import jax
import jax.numpy as jnp
from jax import lax
from jax.experimental import pallas as pl
from jax.experimental.pallas import tpu as pltpu
from jax.experimental.pallas import tpu_sc as plsc

BATCH = 16384
DF = 64
DW = 128

_NC = 2
_NS = 16
_NW = _NC * _NS
_BPW = BATCH // _NW
_CH = 128
_NCH = _BPW // _CH
_HALF = _BPW // 2


_RBLK = 2000


def _repack_body(nblk, in_hbm, out_ref, buf, sem):
    i = pl.program_id(0)

    def start(k, slot):
        return pltpu.make_async_copy(
            in_hbm.at[pl.ds(k * 2 * _RBLK, 2 * _RBLK), :],
            buf.at[slot], sem).start()

    @pl.when(i == 0)
    def _():
        start(0, 0)

    @pl.when(i + 1 < nblk)
    def _():
        start(i + 1, (i + 1) % 2)

    pltpu.make_async_copy(
        in_hbm.at[pl.ds(i * 2 * _RBLK, 2 * _RBLK), :],
        buf.at[i % 2], sem).wait()
    x = buf[i % 2].reshape(_RBLK, 2, DF)
    out_ref[...] = jnp.concatenate([x[:, 0, :], x[:, 1, :]], axis=-1)


def _repack(table):
    n2 = table.shape[0] // 2
    nblk = n2 // _RBLK
    return pl.pallas_call(
        functools.partial(_repack_body, nblk),
        grid=(nblk,),
        in_specs=[pl.BlockSpec(memory_space=pl.ANY)],
        out_specs=pl.BlockSpec((_RBLK, DW), lambda i: (i, 0)),
        out_shape=jax.ShapeDtypeStruct((n2, DW), jnp.float32),
        scratch_shapes=[
            pltpu.VMEM((2, 2 * _RBLK, DF), jnp.float32),
            pltpu.SemaphoreType.DMA,
        ],
    )(table)


@functools.cache
def _make_sc_gather():
    mesh = plsc.VectorSubcoreMesh(core_axis_name="c", subcore_axis_name="s")

    def _body(idx_hbm, tab_hbm, out_hbm, idx_v, rows_v, gsem, wsem):
        wid = lax.axis_index("s") * _NC + lax.axis_index("c")
        base = wid * _BPW
        pltpu.sync_copy(idx_hbm.at[pl.ds(wid * _NCH, _NCH)], idx_v)
        writes = []
        for p in range(2):
            if p:
                for w in writes:
                    w.wait()
                writes = []
            gathers = []
            for j in range(2):
                c = 2 * p + j
                gathers.append(pltpu.async_copy(
                    tab_hbm.at[idx_v.at[c]],
                    rows_v.at[pl.ds(j * _CH, _CH)], gsem))
            for g in gathers:
                g.wait()
            writes.append(pltpu.async_copy(
                rows_v, out_hbm.at[pl.ds(base + p * _HALF, _HALF)], wsem))
        for w in writes:
            w.wait()

    return pl.kernel(
        _body,
        mesh=mesh,
        out_type=jax.ShapeDtypeStruct((BATCH, DW), jnp.float32),
        scratch_types=[
            pltpu.VMEM((_NCH, _CH), jnp.int32),
            pltpu.VMEM((_HALF, DW), jnp.float32),
            pltpu.SemaphoreType.DMA,
            pltpu.SemaphoreType.DMA,
        ],
    )


_BLK = 1024


def _mlp_body(u_ref, s_ref, up_ref, sp_ref, w0u_ref, w0s_ref, b0_ref,
              w1_ref, b1_ref, w2_ref, b2_ref, w3_ref, b3_ref, out_ref):
    u2 = u_ref[...]
    s2 = s_ref[...]
    u = jnp.where(up_ref[...] != 0, u2[:, DF:], u2[:, :DF])
    s = jnp.where(sp_ref[...] != 0, s2[:, DF:], s2[:, :DF])
    x = (jnp.dot(u, w0u_ref[...], preferred_element_type=jnp.float32)
         + jnp.dot(s, w0s_ref[...], preferred_element_type=jnp.float32)
         + b0_ref[...])
    x = jnp.maximum(x, 0.0)
    x = jnp.dot(x, w1_ref[...], preferred_element_type=jnp.float32) + b1_ref[...]
    x = jnp.maximum(x, 0.0)
    x = jnp.dot(x, w2_ref[...], preferred_element_type=jnp.float32) + b2_ref[...]
    x = jnp.maximum(x, 0.0)
    out_ref[...] = jnp.sum(x * w3_ref[...], axis=1) + b3_ref[0, 0]


def _mlp(u, s, up, sp, w0u, w0s, b0, w1, b1, w2, b2, w3, b3):
    grid = (BATCH // _BLK,)
    full = lambda shape: pl.BlockSpec(shape, lambda i: (0,) * len(shape))
    return pl.pallas_call(
        _mlp_body,
        grid=grid,
        in_specs=[
            pl.BlockSpec((_BLK, DW), lambda i: (i, 0)),
            pl.BlockSpec((_BLK, DW), lambda i: (i, 0)),
            pl.BlockSpec((_BLK, 1), lambda i: (i, 0)),
            pl.BlockSpec((_BLK, 1), lambda i: (i, 0)),
            full((DF, 128)),
            full((DF, 128)),
            full((1, 128)),
            full((128, 64)),
            full((1, 64)),
            full((64, 32)),
            full((1, 32)),
            full((1, 32)),
            full((1, 1)),
        ],
        out_specs=pl.BlockSpec((_BLK,), lambda i: (i,)),
        out_shape=jax.ShapeDtypeStruct((BATCH,), jnp.float32),
    )(u, s, up, sp, w0u, w0s, b0, w1, b1, w2, b2, w3, b3)


def kernel(user_ids, song_ids, user_table, song_table,
           W0, b0, W1, b1, W2, b2, W3, b3):
    uid = user_ids.astype(jnp.int32)
    sid = song_ids.astype(jnp.int32)
    uidx = (uid >> 1).reshape(_NW * _NCH, _CH)
    sidx = (sid >> 1).reshape(_NW * _NCH, _CH)
    upar = (uid & 1).reshape(BATCH, 1)
    spar = (sid & 1).reshape(BATCH, 1)
    st2 = _repack(song_table)
    ut2 = _repack(user_table)
    gather = _make_sc_gather()
    s = gather(sidx, st2)
    u = gather(uidx, ut2)
    w0t = W0.T
    return _mlp(
        u, s, upar, spar,
        w0t[:DF], w0t[DF:], b0.reshape(1, 128),
        W1.T, b1.reshape(1, 64),
        W2.T, b2.reshape(1, 32),
        W3, b3.reshape(1, 1),
    )


# speedup vs baseline: 1.1030x; 1.0989x over previous
import functools

import jax
import jax.numpy as jnp
from jax import lax
from jax.experimental import pallas as pl
from jax.experimental.pallas import tpu as pltpu
from jax.experimental.pallas import tpu_sc as plsc

BATCH = 16384
DF = 64
DW = 128

_NC = 2
_NS = 16
_NW = _NC * _NS
_BPW = BATCH // _NW
_CH = 128
_NCH = _BPW // _CH
_HALF = _BPW // 2


_RBLK = 5000
_NSLOT = 3


def _repack_body(nblk, in_hbm, out_ref, buf, sems):
    i = pl.program_id(0)

    def start(k):
        return pltpu.make_async_copy(
            in_hbm.at[pl.ds(k * 2 * _RBLK, 2 * _RBLK), :],
            buf.at[k % _NSLOT], sems.at[k % _NSLOT]).start()

    @pl.when(i == 0)
    def _():
        for k in range(_NSLOT - 1):
            start(k)

    @pl.when(i + _NSLOT - 1 < nblk)
    def _():
        start(i + _NSLOT - 1)

    pltpu.make_async_copy(
        in_hbm.at[pl.ds(i * 2 * _RBLK, 2 * _RBLK), :],
        buf.at[i % _NSLOT], sems.at[i % _NSLOT]).wait()
    x = buf[i % _NSLOT].reshape(_RBLK, 2, DF)
    out_ref[...] = jnp.concatenate([x[:, 0, :], x[:, 1, :]], axis=-1)


def _repack(table):
    n2 = table.shape[0] // 2
    nblk = n2 // _RBLK
    return pl.pallas_call(
        functools.partial(_repack_body, nblk),
        grid=(nblk,),
        in_specs=[pl.BlockSpec(memory_space=pl.ANY)],
        out_specs=pl.BlockSpec((_RBLK, DW), lambda i: (i, 0)),
        out_shape=jax.ShapeDtypeStruct((n2, DW), jnp.float32),
        scratch_shapes=[
            pltpu.VMEM((_NSLOT, 2 * _RBLK, DF), jnp.float32),
            pltpu.SemaphoreType.DMA((_NSLOT,)),
        ],
    )(table)


@functools.cache
def _make_sc_gather():
    mesh = plsc.VectorSubcoreMesh(core_axis_name="c", subcore_axis_name="s")

    def _body(idx_hbm, tab_hbm, out_hbm, idx_v, rows_v, gsem, wsem):
        wid = lax.axis_index("s") * _NC + lax.axis_index("c")
        base = wid * _BPW
        pltpu.sync_copy(idx_hbm.at[pl.ds(wid * _NCH, _NCH)], idx_v)
        writes = []
        for p in range(2):
            if p:
                for w in writes:
                    w.wait()
                writes = []
            gathers = []
            for j in range(2):
                c = 2 * p + j
                gathers.append(pltpu.async_copy(
                    tab_hbm.at[idx_v.at[c]],
                    rows_v.at[pl.ds(j * _CH, _CH)], gsem))
            for g in gathers:
                g.wait()
            writes.append(pltpu.async_copy(
                rows_v, out_hbm.at[pl.ds(base + p * _HALF, _HALF)], wsem))
        for w in writes:
            w.wait()

    return pl.kernel(
        _body,
        mesh=mesh,
        out_type=jax.ShapeDtypeStruct((BATCH, DW), jnp.float32),
        scratch_types=[
            pltpu.VMEM((_NCH, _CH), jnp.int32),
            pltpu.VMEM((_HALF, DW), jnp.float32),
            pltpu.SemaphoreType.DMA,
            pltpu.SemaphoreType.DMA,
        ],
    )


_BLK = 1024


def _mlp_body(u_ref, s_ref, up_ref, sp_ref, w0u_ref, w0s_ref, b0_ref,
              w1_ref, b1_ref, w2_ref, b2_ref, w3_ref, b3_ref, out_ref):
    u2 = u_ref[...]
    s2 = s_ref[...]
    u = jnp.where(up_ref[...] != 0, u2[:, DF:], u2[:, :DF])
    s = jnp.where(sp_ref[...] != 0, s2[:, DF:], s2[:, :DF])
    x = (jnp.dot(u, w0u_ref[...], preferred_element_type=jnp.float32)
         + jnp.dot(s, w0s_ref[...], preferred_element_type=jnp.float32)
         + b0_ref[...])
    x = jnp.maximum(x, 0.0)
    x = jnp.dot(x, w1_ref[...], preferred_element_type=jnp.float32) + b1_ref[...]
    x = jnp.maximum(x, 0.0)
    x = jnp.dot(x, w2_ref[...], preferred_element_type=jnp.float32) + b2_ref[...]
    x = jnp.maximum(x, 0.0)
    out_ref[...] = jnp.sum(x * w3_ref[...], axis=1) + b3_ref[0, 0]


def _mlp(u, s, up, sp, w0u, w0s, b0, w1, b1, w2, b2, w3, b3):
    grid = (BATCH // _BLK,)
    full = lambda shape: pl.BlockSpec(shape, lambda i: (0,) * len(shape))
    return pl.pallas_call(
        _mlp_body,
        grid=grid,
        in_specs=[
            pl.BlockSpec((_BLK, DW), lambda i: (i, 0)),
            pl.BlockSpec((_BLK, DW), lambda i: (i, 0)),
            pl.BlockSpec((_BLK, 1), lambda i: (i, 0)),
            pl.BlockSpec((_BLK, 1), lambda i: (i, 0)),
            full((DF, 128)),
            full((DF, 128)),
            full((1, 128)),
            full((128, 64)),
            full((1, 64)),
            full((64, 32)),
            full((1, 32)),
            full((1, 32)),
            full((1, 1)),
        ],
        out_specs=pl.BlockSpec((_BLK,), lambda i: (i,)),
        out_shape=jax.ShapeDtypeStruct((BATCH,), jnp.float32),
    )(u, s, up, sp, w0u, w0s, b0, w1, b1, w2, b2, w3, b3)


def kernel(user_ids, song_ids, user_table, song_table,
           W0, b0, W1, b1, W2, b2, W3, b3):
    uid = user_ids.astype(jnp.int32)
    sid = song_ids.astype(jnp.int32)
    uidx = (uid >> 1).reshape(_NW * _NCH, _CH)
    sidx = (sid >> 1).reshape(_NW * _NCH, _CH)
    upar = (uid & 1).reshape(BATCH, 1)
    spar = (sid & 1).reshape(BATCH, 1)
    st2 = _repack(song_table)
    ut2 = _repack(user_table)
    gather = _make_sc_gather()
    s = gather(sidx, st2)
    u = gather(uidx, ut2)
    w0t = W0.T
    return _mlp(
        u, s, upar, spar,
        w0t[:DF], w0t[DF:], b0.reshape(1, 128),
        W1.T, b1.reshape(1, 64),
        W2.T, b2.reshape(1, 32),
        W3, b3.reshape(1, 1),
    )
